# baseline (speedup 1.0000x reference)
import functools

import jax
import jax.numpy as jnp
from jax import lax
from jax.experimental import pallas as pl
from jax.experimental.pallas import tpu as pltpu

HIDDEN = 2048
INTER = 1408
E = 8
TOP_K = 2
N_GROUP = 2
TOPK_GROUP = 1
SCALE = 2.5
T = 2048

BT = 256
NT = T // BT
NI = 11
BI = INTER // NI
GSZ = E // N_GROUP


def _router_body(x_ref, gw_ref, bias_ref, w_ref):
    xb = x_ref[...]
    gw = gw_ref[...]
    logits = lax.dot_general(xb, gw, (((1,), (1,)), ((), ())),
                             preferred_element_type=jnp.float32)
    s = jax.nn.sigmoid(logits)
    sc = s + bias_ref[...]

    lane = lax.broadcasted_iota(jnp.int32, (BT, E), 1)
    gid = lane // GSZ

    a = sc[:, :, None]
    b = sc[:, None, :]
    i_idx = lax.broadcasted_iota(jnp.int32, (E, E), 0)[None]
    j_idx = lax.broadcasted_iota(jnp.int32, (E, E), 1)[None]
    same_group = (i_idx // GSZ) == (j_idx // GSZ)
    beats = (b > a) | ((b == a) & (j_idx < i_idx))
    cnt_in_group = jnp.sum(
        jnp.where(same_group & beats, 1, 0).astype(jnp.int32), axis=2)
    in_top2 = cnt_in_group < 2
    contrib = jnp.where(in_top2, sc, 0.0)
    t2s0 = jnp.sum(jnp.where(gid == 0, contrib, 0.0), axis=1, keepdims=True)
    t2s1 = jnp.sum(jnp.where(gid == 1, contrib, 0.0), axis=1, keepdims=True)
    gsel = jnp.where(t2s0 >= t2s1, 0, 1)

    scm = jnp.where(gid == gsel, sc, 0.0)
    am = scm[:, :, None]
    bm = scm[:, None, :]
    beats_m = (bm > am) | ((bm == am) & (j_idx < i_idx))
    cnt_all = jnp.sum(beats_m.astype(jnp.int32), axis=2)
    sel = cnt_all < TOP_K

    w_raw = jnp.where(sel, s, 0.0)
    denom = jnp.sum(w_raw, axis=1, keepdims=True) + 1e-20
    w_ref[...] = w_raw / denom * SCALE


def _expert_body(x_ref, wcomb_ref, gate_ref, up_ref, down_ref,
                 shg_ref, shu_ref, shd_ref, out_ref):
    e = pl.program_id(0)
    i = pl.program_id(1)
    t = pl.program_id(2)

    @pl.when((e == 0) & (i == 0) & (t == 0))
    def _():
        out_ref[...] = jnp.zeros_like(out_ref)

    xb = x_ref[pl.ds(t * BT, BT), :]

    def mlp(g_w, u_w, d_w):
        g = lax.dot_general(xb, g_w, (((1,), (1,)), ((), ())),
                            preferred_element_type=jnp.float32)
        u = lax.dot_general(xb, u_w, (((1,), (1,)), ((), ())),
                            preferred_element_type=jnp.float32)
        h = g * jax.nn.sigmoid(g) * u
        return lax.dot_general(h, d_w, (((1,), (1,)), ((), ())),
                               preferred_element_type=jnp.float32)

    @pl.when(e < E)
    def _():
        ye = mlp(gate_ref[0], up_ref[0], down_ref[0])
        onehot = (lax.broadcasted_iota(jnp.int32, (BT, E), 1) == e)
        wcol = jnp.sum(jnp.where(onehot, wcomb_ref[...], 0.0),
                       axis=1, keepdims=True)
        out_ref[pl.ds(t * BT, BT), :] += ye * wcol

    @pl.when(e >= E)
    def _():
        ye = mlp(shg_ref[...], shu_ref[...], shd_ref[...])
        out_ref[pl.ds(t * BT, BT), :] += ye


@jax.jit
def kernel(hidden_states, gate_w, e_score_correction_bias,
           w_gate_up, w_down, sh_gate_up, sh_down):
    x = hidden_states
    bias2d = e_score_correction_bias.reshape(1, E)

    wcomb = pl.pallas_call(
        _router_body,
        grid=(NT,),
        in_specs=[
            pl.BlockSpec((BT, HIDDEN), lambda t: (t, 0)),
            pl.BlockSpec((E, HIDDEN), lambda t: (0, 0)),
            pl.BlockSpec((1, E), lambda t: (0, 0)),
        ],
        out_specs=pl.BlockSpec((BT, E), lambda t: (t, 0)),
        out_shape=jax.ShapeDtypeStruct((T, E), jnp.float32),
    )(x, gate_w, bias2d)

    def e7(e):
        return jnp.minimum(e, E - 1)

    out = pl.pallas_call(
        _expert_body,
        grid=(E + 2, NI, NT),
        in_specs=[
            pl.BlockSpec((T, HIDDEN), lambda e, i, t: (0, 0)),
            pl.BlockSpec((BT, E), lambda e, i, t: (t, 0)),
            pl.BlockSpec((1, BI, HIDDEN), lambda e, i, t: (e7(e), i, 0)),
            pl.BlockSpec((1, BI, HIDDEN), lambda e, i, t: (e7(e), NI + i, 0)),
            pl.BlockSpec((1, HIDDEN, BI), lambda e, i, t: (e7(e), 0, i)),
            pl.BlockSpec((BI, HIDDEN),
                         lambda e, i, t: (jnp.where(e < E, 0,
                                                    NI * (e - E) + i), 0)),
            pl.BlockSpec((BI, HIDDEN),
                         lambda e, i, t: (jnp.where(e < E, 2 * NI,
                                                    2 * NI + NI * (e - E) + i),
                                          0)),
            pl.BlockSpec((HIDDEN, BI),
                         lambda e, i, t: (0, jnp.where(e < E, 0,
                                                       NI * (e - E) + i))),
        ],
        out_specs=pl.BlockSpec((T, HIDDEN), lambda e, i, t: (0, 0)),
        out_shape=jax.ShapeDtypeStruct((T, HIDDEN), jnp.float32),
    )(x, wcomb, w_gate_up, w_gate_up, w_down, sh_gate_up, sh_gate_up, sh_down)

    return out
